# baseline (speedup 1.0000x reference)
import functools

import jax
import jax.numpy as jnp
from jax.experimental import pallas as pl
from jax.experimental.pallas import tpu as pltpu

D_IN, H0, H1, D_OUT, B = 2048, 4096, 2048, 32768, 128


def _mm_kernel(x_ref, w_ref, b_ref, o_ref, *, act):
    acc = jnp.dot(x_ref[...], w_ref[...], preferred_element_type=jnp.float32)
    acc = acc + b_ref[...]
    if act == "relu":
        acc = jnp.maximum(acc, 0.0)
    elif act == "sigmoid":
        acc = 1.0 / (1.0 + jnp.exp(-acc))
    o_ref[...] = acc


def _matmul(x, w, b2d, act, blk_n):
    m, k = x.shape
    n = w.shape[1]
    grid = n // blk_n
    return pl.pallas_call(
        functools.partial(_mm_kernel, act=act),
        grid=(grid,),
        in_specs=[
            pl.BlockSpec((m, k), lambda i: (0, 0)),
            pl.BlockSpec((k, blk_n), lambda i: (0, i)),
            pl.BlockSpec((1, blk_n), lambda i: (0, i)),
        ],
        out_specs=pl.BlockSpec((m, blk_n), lambda i: (0, i)),
        out_shape=jax.ShapeDtypeStruct((m, n), jnp.float32),
    )(x, w, b2d)


def _mask_kernel(probs_ref, x_ref, ws1_ref, bs1_ref, ws2_ref, bs2_ref, out_ref):
    rows = probs_ref.shape[0]
    s1 = jnp.maximum(
        jnp.dot(x_ref[...], ws1_ref[...], preferred_element_type=jnp.float32)
        + bs1_ref[...],
        0.0,
    )
    s2 = jnp.dot(s1, ws2_ref[...], preferred_element_type=jnp.float32) + bs2_ref[...]
    sf = 1.0 / (1.0 + jnp.exp(-s2))
    rate = 0.005 + sf * 0.095
    k = jnp.maximum(1, jnp.floor(D_OUT * rate).astype(jnp.int32))

    bits = jax.lax.bitcast_convert_type(probs_ref[...], jnp.int32)

    hi16 = (bits >> 15).astype(jnp.int16)
    low16 = (bits & 0x7FFF).astype(jnp.int16)

    n_chunks = 16
    cw = D_OUT // n_chunks

    def _tree_sum(parts):
        while len(parts) > 1:
            parts = [
                parts[i] + parts[i + 1] if i + 1 < len(parts) else parts[i]
                for i in range(0, len(parts), 2)
            ]
        return parts[0]

    def count_gt16(arr, t16):
        acc = _tree_sum(
            [
                (arr[:, c * cw : (c + 1) * cw] > t16).astype(jnp.int16)
                for c in range(n_chunks)
            ]
        )
        return jnp.sum(acc.astype(jnp.int32), axis=1, keepdims=True)

    def body_a(_, lohi):
        lo, hi = lohi
        mid = (lo + hi) >> 1
        small = count_gt16(hi16, mid.astype(jnp.int16)) < k
        return jnp.where(small, lo, mid + 1), jnp.where(small, mid, hi)

    lo0 = jnp.zeros((rows, 1), jnp.int32)
    hi0 = jnp.full((rows, 1), 0x7F00, jnp.int32)
    _, tau16 = jax.lax.fori_loop(0, 15, body_a, (lo0, hi0))
    t16 = tau16.astype(jnp.int16)

    w16 = jnp.where(hi16 == t16, low16, jnp.int16(-1))

    def body_b(_, lohi):
        lo, hi = lohi
        mid = (lo + hi) >> 1
        small = count_gt16(w16, mid.astype(jnp.int16)) < k - c_hi
        return jnp.where(small, lo, mid + 1), jnp.where(small, mid, hi)

    c_hi = count_gt16(hi16, t16)
    lo0 = jnp.zeros((rows, 1), jnp.int32)
    hi0 = jnp.full((rows, 1), 0x7FFF, jnp.int32)
    _, tau_low = jax.lax.fori_loop(0, 15, body_b, (lo0, hi0))

    tau = (tau16 << 15) | tau_low
    need = k - c_hi - count_gt16(w16, tau_low.astype(jnp.int16))

    cols = jax.lax.broadcasted_iota(jnp.int32, (rows, D_OUT), 1)
    eq16 = (hi16 == t16) & (low16 == tau_low.astype(jnp.int16))
    v16 = jnp.where(eq16, cols.astype(jnp.int16), jnp.int16(32767))

    def body2(_, lohi):
        lo, hi = lohi
        mid = (lo + hi) >> 1
        ge = count_gt16(v16, mid.astype(jnp.int16) - 1) <= D_OUT - need
        return jnp.where(ge, lo, mid + 1), jnp.where(ge, mid, hi)

    lo0 = jnp.zeros((rows, 1), jnp.int32)
    hi0 = jnp.full((rows, 1), D_OUT, jnp.int32)
    _, cstar = jax.lax.fori_loop(0, 16, body2, (lo0, hi0))

    greater = bits > tau
    eq = bits == tau
    out_ref[...] = (greater | (eq & (cols < cstar))).astype(jnp.float32)


def kernel(x, W1, b1, W2, b2, W3, b3, Ws1, bs1, Ws2, bs2):
    h1 = _matmul(x, W1.astype(jnp.bfloat16), b1.reshape(1, H0), "relu", 512)
    h2 = _matmul(h1, W2.astype(jnp.bfloat16), b2.reshape(1, H1), "relu", 256)
    probs = _matmul(h2, W3.astype(jnp.bfloat16), b3.reshape(1, D_OUT), "sigmoid", 1024)

    blk_rows = 32
    mask = pl.pallas_call(
        _mask_kernel,
        grid=(B // blk_rows,),
        in_specs=[
            pl.BlockSpec((blk_rows, D_OUT), lambda i: (i, 0)),
            pl.BlockSpec((blk_rows, D_IN), lambda i: (i, 0)),
            pl.BlockSpec((D_IN, 128), lambda i: (0, 0)),
            pl.BlockSpec((1, 128), lambda i: (0, 0)),
            pl.BlockSpec((128, 1), lambda i: (0, 0)),
            pl.BlockSpec((1, 1), lambda i: (0, 0)),
        ],
        out_specs=pl.BlockSpec((blk_rows, D_OUT), lambda i: (i, 0)),
        out_shape=jax.ShapeDtypeStruct((B, D_OUT), jnp.float32),
    )(probs, x, Ws1, bs1.reshape(1, 128), Ws2, bs2.reshape(1, 1))
    return mask


# speedup vs baseline: 1.5043x; 1.5043x over previous
import functools

import jax
import jax.numpy as jnp
from jax.experimental import pallas as pl
from jax.experimental.pallas import tpu as pltpu

D_IN, H0, H1, D_OUT, B = 2048, 4096, 2048, 32768, 128


def _mm_kernel(x_ref, w_ref, b_ref, o_ref, *, act):
    acc = jnp.dot(x_ref[...], w_ref[...], preferred_element_type=jnp.float32)
    acc = acc + b_ref[...]
    if act == "relu":
        acc = jnp.maximum(acc, 0.0)
    elif act == "sigmoid":
        acc = 1.0 / (1.0 + jnp.exp(-acc))
    o_ref[...] = acc


def _matmul(x, w, b2d, act, blk_n):
    m, k = x.shape
    n = w.shape[1]
    grid = n // blk_n
    return pl.pallas_call(
        functools.partial(_mm_kernel, act=act),
        grid=(grid,),
        in_specs=[
            pl.BlockSpec((m, k), lambda i: (0, 0)),
            pl.BlockSpec((k, blk_n), lambda i: (0, i)),
            pl.BlockSpec((1, blk_n), lambda i: (0, i)),
        ],
        out_specs=pl.BlockSpec((m, blk_n), lambda i: (0, i)),
        out_shape=jax.ShapeDtypeStruct((m, n), jnp.float32),
    )(x, w, b2d)


def _mask_kernel(probs_ref, x_ref, ws1_ref, bs1_ref, ws2_ref, bs2_ref, out_ref):
    rows = probs_ref.shape[0]
    s1 = jnp.maximum(
        jnp.dot(x_ref[...], ws1_ref[...], preferred_element_type=jnp.float32)
        + bs1_ref[...],
        0.0,
    )
    s2 = jnp.dot(s1, ws2_ref[...], preferred_element_type=jnp.float32) + bs2_ref[...]
    sf = 1.0 / (1.0 + jnp.exp(-s2))
    rate = 0.005 + sf * 0.095
    k = jnp.maximum(1, jnp.floor(D_OUT * rate).astype(jnp.int32))

    bits = jax.lax.bitcast_convert_type(probs_ref[...], jnp.int32)

    hi16 = (bits >> 15).astype(jnp.int16)
    low16 = (bits & 0x7FFF).astype(jnp.int16)

    n_chunks = 16
    cw = D_OUT // n_chunks

    def _tree_sum(parts):
        while len(parts) > 1:
            parts = [
                parts[i] + parts[i + 1] if i + 1 < len(parts) else parts[i]
                for i in range(0, len(parts), 2)
            ]
        return parts[0]

    def count_gt16(arr, t16):
        acc = _tree_sum(
            [
                (arr[:, c * cw : (c + 1) * cw] > t16).astype(jnp.int16)
                for c in range(n_chunks)
            ]
        )
        return jnp.sum(acc.astype(jnp.int32), axis=1, keepdims=True)

    def body_a(_, lohi):
        lo, hi = lohi
        mid = (lo + hi) >> 1
        small = count_gt16(hi16, mid.astype(jnp.int16)) < k
        return jnp.where(small, lo, mid + 1), jnp.where(small, mid, hi)

    lo0 = jnp.zeros((rows, 1), jnp.int32)
    hi0 = jnp.full((rows, 1), 0x7F00, jnp.int32)
    _, tau16 = jax.lax.fori_loop(0, 15, body_a, (lo0, hi0))
    t16 = tau16.astype(jnp.int16)

    w16 = jnp.where(hi16 == t16, low16, jnp.int16(-1))

    def body_b(_, lohi):
        lo, hi = lohi
        mid = (lo + hi) >> 1
        small = count_gt16(w16, mid.astype(jnp.int16)) < k - c_hi
        return jnp.where(small, lo, mid + 1), jnp.where(small, mid, hi)

    c_hi = count_gt16(hi16, t16)
    lo0 = jnp.zeros((rows, 1), jnp.int32)
    hi0 = jnp.full((rows, 1), 0x7FFF, jnp.int32)
    _, tau_low = jax.lax.fori_loop(0, 15, body_b, (lo0, hi0))

    tau = (tau16 << 15) | tau_low
    need = k - c_hi - count_gt16(w16, tau_low.astype(jnp.int16))

    cols = jax.lax.broadcasted_iota(jnp.int32, (rows, D_OUT), 1)
    eq16 = (hi16 == t16) & (low16 == tau_low.astype(jnp.int16))
    v16 = jnp.where(eq16, cols.astype(jnp.int16), jnp.int16(32767))

    def body2(_, lohi):
        lo, hi = lohi
        mid = (lo + hi) >> 1
        ge = count_gt16(v16, mid.astype(jnp.int16) - 1) <= D_OUT - need
        return jnp.where(ge, lo, mid + 1), jnp.where(ge, mid, hi)

    lo0 = jnp.zeros((rows, 1), jnp.int32)
    hi0 = jnp.full((rows, 1), D_OUT, jnp.int32)
    _, cstar = jax.lax.fori_loop(0, 16, body2, (lo0, hi0))

    greater = bits > tau
    eq = bits == tau
    out_ref[...] = (greater | (eq & (cols < cstar))).astype(jnp.float32)


def kernel(x, W1, b1, W2, b2, W3, b3, Ws1, bs1, Ws2, bs2):
    h1 = _matmul(x, W1, b1.reshape(1, H0), "relu", 512)
    h2 = _matmul(h1, W2, b2.reshape(1, H1), "relu", 256)
    probs = _matmul(h2, W3, b3.reshape(1, D_OUT), "sigmoid", 1024)

    blk_rows = 32
    mask = pl.pallas_call(
        _mask_kernel,
        grid=(B // blk_rows,),
        in_specs=[
            pl.BlockSpec((blk_rows, D_OUT), lambda i: (i, 0)),
            pl.BlockSpec((blk_rows, D_IN), lambda i: (i, 0)),
            pl.BlockSpec((D_IN, 128), lambda i: (0, 0)),
            pl.BlockSpec((1, 128), lambda i: (0, 0)),
            pl.BlockSpec((128, 1), lambda i: (0, 0)),
            pl.BlockSpec((1, 1), lambda i: (0, 0)),
        ],
        out_specs=pl.BlockSpec((blk_rows, D_OUT), lambda i: (i, 0)),
        out_shape=jax.ShapeDtypeStruct((B, D_OUT), jnp.float32),
    )(probs, x, Ws1, bs1.reshape(1, 128), Ws2, bs2.reshape(1, 1))
    return mask


# speedup vs baseline: 1.5894x; 1.0565x over previous
import functools

import jax
import jax.numpy as jnp
from jax.experimental import pallas as pl
from jax.experimental.pallas import tpu as pltpu

D_IN, H0, H1, D_OUT, B = 2048, 4096, 2048, 32768, 128


def _mm_kernel(x_ref, w_ref, b_ref, o_ref, *, act):
    acc = jnp.dot(x_ref[...], w_ref[...], preferred_element_type=jnp.float32)
    acc = acc + b_ref[...]
    if act == "relu":
        acc = jnp.maximum(acc, 0.0)
    elif act == "sigmoid":
        acc = 1.0 / (1.0 + jnp.exp(-acc))
    o_ref[...] = acc


def _matmul(x, w, b2d, act, blk_n):
    m, k = x.shape
    n = w.shape[1]
    grid = n // blk_n
    return pl.pallas_call(
        functools.partial(_mm_kernel, act=act),
        grid=(grid,),
        in_specs=[
            pl.BlockSpec((m, k), lambda i: (0, 0)),
            pl.BlockSpec((k, blk_n), lambda i: (0, i)),
            pl.BlockSpec((1, blk_n), lambda i: (0, i)),
        ],
        out_specs=pl.BlockSpec((m, blk_n), lambda i: (0, i)),
        out_shape=jax.ShapeDtypeStruct((m, n), jnp.float32),
    )(x, w, b2d)


def _mask_kernel(probs_ref, x_ref, ws1_ref, bs1_ref, ws2_ref, bs2_ref, out_ref):
    rows = probs_ref.shape[0]
    s1 = jnp.maximum(
        jnp.dot(x_ref[...], ws1_ref[...], preferred_element_type=jnp.float32)
        + bs1_ref[...],
        0.0,
    )
    s2 = jnp.dot(s1, ws2_ref[...], preferred_element_type=jnp.float32) + bs2_ref[...]
    sf = 1.0 / (1.0 + jnp.exp(-s2))
    rate = 0.005 + sf * 0.095
    k = jnp.maximum(1, jnp.floor(D_OUT * rate).astype(jnp.int32))

    bits = jax.lax.bitcast_convert_type(probs_ref[...], jnp.int32)

    hi16 = (bits >> 15).astype(jnp.int16)
    low16 = (bits & 0x7FFF).astype(jnp.int16)

    n_chunks = 16
    cw = D_OUT // n_chunks

    def _tree_sum(parts):
        while len(parts) > 1:
            parts = [
                parts[i] + parts[i + 1] if i + 1 < len(parts) else parts[i]
                for i in range(0, len(parts), 2)
            ]
        return parts[0]

    def count_gt16(arr, t16):
        acc = _tree_sum(
            [
                (arr[:, c * cw : (c + 1) * cw] > t16).astype(jnp.int16)
                for c in range(n_chunks)
            ]
        )
        return jnp.sum(acc.astype(jnp.int32), axis=1, keepdims=True)

    def body_a(_, lohi):
        lo, hi = lohi
        mid = (lo + hi) >> 1
        small = count_gt16(hi16, mid.astype(jnp.int16)) < k
        return jnp.where(small, lo, mid + 1), jnp.where(small, mid, hi)

    lo0 = jnp.zeros((rows, 1), jnp.int32)
    hi0 = jnp.full((rows, 1), 0x7F00, jnp.int32)
    _, tau16 = jax.lax.fori_loop(0, 15, body_a, (lo0, hi0))
    t16 = tau16.astype(jnp.int16)

    w16 = jnp.where(hi16 == t16, low16, jnp.int16(-1))

    def body_b(_, lohi):
        lo, hi = lohi
        mid = (lo + hi) >> 1
        small = count_gt16(w16, mid.astype(jnp.int16)) < k - c_hi
        return jnp.where(small, lo, mid + 1), jnp.where(small, mid, hi)

    c_hi = count_gt16(hi16, t16)
    lo0 = jnp.zeros((rows, 1), jnp.int32)
    hi0 = jnp.full((rows, 1), 0x7FFF, jnp.int32)
    _, tau_low = jax.lax.fori_loop(0, 15, body_b, (lo0, hi0))

    tau = (tau16 << 15) | tau_low
    need = k - c_hi - count_gt16(w16, tau_low.astype(jnp.int16))

    cols = jax.lax.broadcasted_iota(jnp.int32, (rows, D_OUT), 1)
    eq16 = (hi16 == t16) & (low16 == tau_low.astype(jnp.int16))
    v16 = jnp.where(eq16, cols.astype(jnp.int16), jnp.int16(32767))

    def body2(_, lohi):
        lo, hi = lohi
        mid = (lo + hi) >> 1
        ge = count_gt16(v16, mid.astype(jnp.int16) - 1) <= D_OUT - need
        return jnp.where(ge, lo, mid + 1), jnp.where(ge, mid, hi)

    lo0 = jnp.zeros((rows, 1), jnp.int32)
    hi0 = jnp.full((rows, 1), D_OUT, jnp.int32)
    _, cstar = jax.lax.fori_loop(0, 16, body2, (lo0, hi0))

    greater = bits > tau
    eq = bits == tau
    out_ref[...] = (greater | (eq & (cols < cstar))).astype(jnp.float32)


def kernel(x, W1, b1, W2, b2, W3, b3, Ws1, bs1, Ws2, bs2):
    h1 = _matmul(x, W1, b1.reshape(1, H0), "relu", 512)
    h2 = _matmul(h1, W2, b2.reshape(1, H1), "relu", 256)
    probs = _matmul(h2, W3, b3.reshape(1, D_OUT), "sigmoid", 1024)

    blk_rows = 64
    mask = pl.pallas_call(
        _mask_kernel,
        grid=(B // blk_rows,),
        in_specs=[
            pl.BlockSpec((blk_rows, D_OUT), lambda i: (i, 0)),
            pl.BlockSpec((blk_rows, D_IN), lambda i: (i, 0)),
            pl.BlockSpec((D_IN, 128), lambda i: (0, 0)),
            pl.BlockSpec((1, 128), lambda i: (0, 0)),
            pl.BlockSpec((128, 1), lambda i: (0, 0)),
            pl.BlockSpec((1, 1), lambda i: (0, 0)),
        ],
        out_specs=pl.BlockSpec((blk_rows, D_OUT), lambda i: (i, 0)),
        out_shape=jax.ShapeDtypeStruct((B, D_OUT), jnp.float32),
    )(probs, x, Ws1, bs1.reshape(1, 128), Ws2, bs2.reshape(1, 1))
    return mask
